# baseline (speedup 1.0000x reference)
import functools

import jax
import jax.numpy as jnp
from jax import lax
from jax.experimental import pallas as pl
from jax.experimental.pallas import tpu as pltpu
from jax.experimental.pallas import tpu_sc as plsc

N = 10000
E = 320000
D = 128
DE = 4

NC = 2
NS = 16
NW = NC * NS
EPT = E // NW
CHUNK = 40
NCH = EPT // CHUNK
ROWS_PT = 624
RB = 1000
EB = 8000


def _proj_body(x_ref, w1_ref, w2_ref, b_ref, p1_ref, p2_ref):
    xb = x_ref[...]
    p1_ref[...] = jnp.dot(xb, w1_ref[...], preferred_element_type=jnp.float32) + b_ref[...]
    p2_ref[...] = jnp.dot(xb, w2_ref[...], preferred_element_type=jnp.float32)


def _update_body(x_ref, p_ref, wu1_ref, wu2_ref, b_ref, o_ref):
    xb = x_ref[...]
    agg = p_ref[0] + p_ref[1]
    h = (jnp.dot(xb, wu1_ref[...], preferred_element_type=jnp.float32)
         + jnp.dot(agg, wu2_ref[...], preferred_element_type=jnp.float32)
         + b_ref[...])
    o_ref[...] = xb + jnp.maximum(h, 0.0)


def _sc_edges_body(p1_hbm, p2_hbm, idx_hbm, ea_hbm, w3_hbm, out_hbm,
                   i0, i1, a0, b0, c0, a1, b1, c1, w3_v, agg,
                   sg0, sg1, si0, si1):
    cid = lax.axis_index("c")
    sid = lax.axis_index("s")
    wid = cid * NS + sid

    pltpu.sync_copy(w3_hbm, w3_v)

    def _zrow(r, carry):
        for d in range(D // 16):
            a0[r, pl.ds(d * 16, 16)] = jnp.zeros((16,), jnp.float32)
        return carry
    lax.fori_loop(0, CHUNK, _zrow, 0)
    base = sid * ROWS_PT
    for k in range(ROWS_PT // CHUNK):
        pltpu.sync_copy(a0, agg.at[pl.ds(base + k * CHUNK, CHUNK)])
    rem = ROWS_PT % CHUNK
    if rem:
        pltpu.sync_copy(a0.at[pl.ds(0, rem)],
                        agg.at[pl.ds(base + ROWS_PT - rem, rem)])

    @pl.when(sid == NS - 1)
    def _zero_tail():
        pltpu.sync_copy(a0.at[pl.ds(0, N - NS * ROWS_PT)],
                        agg.at[pl.ds(NS * ROWS_PT, N - NS * ROWS_PT)])
    plsc.subcore_barrier()

    def _process(bufA, bufB, bufE):
        def _grp(g, carry):
            ev = bufE[pl.ds(16 * g, 16)]
            for jj in range(4):
                e = 4 * g + jj
                s0 = jnp.full((16,), ev[4 * jj], jnp.float32)
                s1 = jnp.full((16,), ev[4 * jj + 1], jnp.float32)
                s2 = jnp.full((16,), ev[4 * jj + 2], jnp.float32)
                s3 = jnp.full((16,), ev[4 * jj + 3], jnp.float32)
                for d in range(D // 16):
                    sl = pl.ds(d * 16, 16)
                    v = bufA[e, sl] + bufB[e, sl]
                    v = v + s0 * w3_v[0, sl] + s1 * w3_v[1, sl]
                    v = v + s2 * w3_v[2, sl] + s3 * w3_v[3, sl]
                    bufA[e, sl] = jnp.maximum(v, 0.0)
            return carry
        lax.fori_loop(0, CHUNK // 4, _grp, 0)

    slots = ((i0, a0, b0, c0, sg0, si0), (i1, a1, b1, c1, sg1, si1))

    def _issue_gathers(j, s):
        iv, bufA, bufB, bufE, sg, _ = slots[s]
        pltpu.async_copy(p1_hbm.at[iv.at[0]], bufA, sg)
        pltpu.async_copy(p2_hbm.at[iv.at[1]], bufB, sg)
        pltpu.async_copy(
            ea_hbm.at[pl.ds((wid * EPT + j * CHUNK) * DE, CHUNK * DE)],
            bufE, sg)

    def _handle(j, s, prefetch_idx, issue_next):
        iv, bufA, bufB, bufE, sg, si = slots[s]
        ivn, _, _, _, _, sin = slots[1 - s]
        pltpu.make_async_copy(p1_hbm.at[iv.at[0]], bufA, sg).wait()
        pltpu.make_async_copy(p2_hbm.at[iv.at[1]], bufB, sg).wait()
        pltpu.make_async_copy(ea_hbm.at[pl.ds(0, CHUNK * DE)],
                              bufE, sg).wait()
        if issue_next:
            pltpu.make_async_copy(idx_hbm.at[wid, 0], ivn, sin).wait()
            _issue_gathers(j + 1, 1 - s)
        _process(bufA, bufB, bufE)
        pltpu.sync_copy(bufA, agg.at[iv.at[1]], add=True)
        if prefetch_idx:
            pltpu.async_copy(idx_hbm.at[wid, j + 2], iv, si)

    pltpu.sync_copy(idx_hbm.at[wid, 0], i0)
    _issue_gathers(0, 0)
    pltpu.async_copy(idx_hbm.at[wid, 1], i1, si1)

    def _pair(i, carry):
        j0 = 2 * i
        _handle(j0, 0, True, True)
        _handle(j0 + 1, 1, True, True)
        return carry
    lax.fori_loop(0, NCH // 2 - 1, _pair, 0)

    _handle(NCH - 2, 0, False, True)
    _handle(NCH - 1, 1, False, False)

    plsc.subcore_barrier()
    pltpu.sync_copy(agg.at[pl.ds(base, ROWS_PT)],
                    out_hbm.at[cid, pl.ds(base, ROWS_PT)])

    @pl.when(sid == NS - 1)
    def _flush_tail():
        pltpu.sync_copy(agg.at[pl.ds(NS * ROWS_PT, N - NS * ROWS_PT)],
                        out_hbm.at[cid, pl.ds(NS * ROWS_PT, N - NS * ROWS_PT)])


_sc_edges = functools.partial(
    pl.kernel,
    out_type=jax.ShapeDtypeStruct((NC, N, D), jnp.float32),
    mesh=plsc.VectorSubcoreMesh(core_axis_name="c", subcore_axis_name="s"),
    scratch_types=[
        pltpu.VMEM((2, CHUNK), jnp.int32),
        pltpu.VMEM((2, CHUNK), jnp.int32),
        pltpu.VMEM((CHUNK, D), jnp.float32),
        pltpu.VMEM((CHUNK, D), jnp.float32),
        pltpu.VMEM((CHUNK * DE,), jnp.float32),
        pltpu.VMEM((CHUNK, D), jnp.float32),
        pltpu.VMEM((CHUNK, D), jnp.float32),
        pltpu.VMEM((CHUNK * DE,), jnp.float32),
        pltpu.VMEM((DE, D), jnp.float32),
        pltpu.VMEM_SHARED((N, D), jnp.float32),
        pltpu.SemaphoreType.DMA,
        pltpu.SemaphoreType.DMA,
        pltpu.SemaphoreType.DMA,
        pltpu.SemaphoreType.DMA,
    ],
)(_sc_edges_body)


def kernel(x, edge_index, edge_attr, W_msg, b_msg, W_upd, b_upd):
    W1 = W_msg[:D]
    W2 = W_msg[D:2 * D]
    W3 = W_msg[2 * D:]
    Wu1 = W_upd[:D]
    Wu2 = W_upd[D:]

    p1, p2 = pl.pallas_call(
        _proj_body,
        grid=(N // RB,),
        in_specs=[
            pl.BlockSpec((RB, D), lambda i: (i, 0)),
            pl.BlockSpec((D, D), lambda i: (0, 0)),
            pl.BlockSpec((D, D), lambda i: (0, 0)),
            pl.BlockSpec((1, D), lambda i: (0, 0)),
        ],
        out_specs=[
            pl.BlockSpec((RB, D), lambda i: (i, 0)),
            pl.BlockSpec((RB, D), lambda i: (i, 0)),
        ],
        out_shape=[
            jax.ShapeDtypeStruct((N, D), jnp.float32),
            jax.ShapeDtypeStruct((N, D), jnp.float32),
        ],
    )(x, W1, W2, b_msg.reshape(1, D))

    idx3d = jnp.stack(
        [edge_index[0].reshape(NW, NCH, CHUNK),
         edge_index[1].reshape(NW, NCH, CHUNK)], axis=2)

    parts = _sc_edges(p1, p2, idx3d, edge_attr.reshape(-1), W3)

    out = pl.pallas_call(
        _update_body,
        grid=(N // RB,),
        in_specs=[
            pl.BlockSpec((RB, D), lambda i: (i, 0)),
            pl.BlockSpec((NC, RB, D), lambda i: (0, i, 0)),
            pl.BlockSpec((D, D), lambda i: (0, 0)),
            pl.BlockSpec((D, D), lambda i: (0, 0)),
            pl.BlockSpec((1, D), lambda i: (0, 0)),
        ],
        out_specs=pl.BlockSpec((RB, D), lambda i: (i, 0)),
        out_shape=jax.ShapeDtypeStruct((N, D), jnp.float32),
    )(x, parts, Wu1, Wu2, b_upd.reshape(1, D))
    return out


# speedup vs baseline: 2.2338x; 2.2338x over previous
import functools

import jax
import jax.numpy as jnp
from jax import lax
from jax.experimental import pallas as pl
from jax.experimental.pallas import tpu as pltpu
from jax.experimental.pallas import tpu_sc as plsc

N = 10000
E = 320000
D = 128
DE = 4

NC = 2
NS = 16
NW = NC * NS
EPT = E // NW
CHUNK = 40
NCH = EPT // CHUNK
ROWS_PT = 624
RB = 1000
EB = 8000


def _proj_body(x_ref, w1_ref, w2_ref, b_ref, p1_ref, p2_ref):
    xb = x_ref[...]
    p1_ref[...] = (jnp.dot(xb, w1_ref[...], preferred_element_type=jnp.float32)
                   + b_ref[...][None, :])
    p2_ref[...] = jnp.dot(xb, w2_ref[...], preferred_element_type=jnp.float32)


def _edge_term_body(ea_ref, w3_ref, c_ref):
    c_ref[...] = jnp.dot(ea_ref[...], w3_ref[0:DE, :],
                         preferred_element_type=jnp.float32)


def _update_body(x_ref, p_ref, wu1_ref, wu2_ref, b_ref, o_ref):
    xb = x_ref[...]
    agg = p_ref[0] + p_ref[1]
    h = (jnp.dot(xb, wu1_ref[...], preferred_element_type=jnp.float32)
         + jnp.dot(agg, wu2_ref[...], preferred_element_type=jnp.float32)
         + b_ref[...][None, :])
    o_ref[...] = xb + jnp.maximum(h, 0.0)


def _sc_edges_body(p1_hbm, p2_hbm, idx_hbm, c_hbm, out_hbm,
                   i0, i1, a0, b0, c0, a1, b1, c1, agg,
                   sg0, sg1, si0, si1):
    cid = lax.axis_index("c")
    sid = lax.axis_index("s")
    wid = cid * NS + sid

    def _zrow(r, carry):
        for d in range(D // 16):
            a0[r, pl.ds(d * 16, 16)] = jnp.zeros((16,), jnp.float32)
        return carry
    lax.fori_loop(0, CHUNK, _zrow, 0)
    base = sid * ROWS_PT
    for k in range(ROWS_PT // CHUNK):
        pltpu.sync_copy(a0, agg.at[pl.ds(base + k * CHUNK, CHUNK)])
    rem = ROWS_PT % CHUNK
    if rem:
        pltpu.sync_copy(a0.at[pl.ds(0, rem)],
                        agg.at[pl.ds(base + ROWS_PT - rem, rem)])

    @pl.when(sid == NS - 1)
    def _zero_tail():
        pltpu.sync_copy(a0.at[pl.ds(0, N - NS * ROWS_PT)],
                        agg.at[pl.ds(NS * ROWS_PT, N - NS * ROWS_PT)])
    plsc.subcore_barrier()

    def _process(bufA, bufB, bufC):
        def _edge(e, carry):
            for d in range(D // 16):
                sl = pl.ds(d * 16, 16)
                v = bufA[e, sl] + bufB[e, sl] + bufC[e, sl]
                bufA[e, sl] = jnp.maximum(v, 0.0)
            return carry
        lax.fori_loop(0, CHUNK, _edge, 0)

    slots = ((i0, a0, b0, c0, sg0, si0), (i1, a1, b1, c1, sg1, si1))

    def _issue_gathers(j, s):
        iv, bufA, bufB, bufC, sg, _ = slots[s]
        pltpu.async_copy(p1_hbm.at[iv.at[0]], bufA, sg)
        pltpu.async_copy(p2_hbm.at[iv.at[1]], bufB, sg)
        pltpu.async_copy(c_hbm.at[pl.ds(wid * EPT + j * CHUNK, CHUNK)],
                         bufC, sg)

    ibase = wid * EPT

    def _idx_issue(j, iv, si):
        pltpu.async_copy(idx_hbm.at[pl.ds(ibase + j * CHUNK, CHUNK)],
                         iv.at[0], si)
        pltpu.async_copy(idx_hbm.at[pl.ds(E + ibase + j * CHUNK, CHUNK)],
                         iv.at[1], si)

    def _idx_drain(iv, si):
        pltpu.make_async_copy(idx_hbm.at[pl.ds(0, CHUNK)], iv.at[0], si).wait()
        pltpu.make_async_copy(idx_hbm.at[pl.ds(0, CHUNK)], iv.at[1], si).wait()

    def _handle(j, s, prefetch_idx, issue_next):
        iv, bufA, bufB, bufC, sg, si = slots[s]
        ivn, _, _, _, _, sin = slots[1 - s]
        pltpu.make_async_copy(p1_hbm.at[iv.at[0]], bufA, sg).wait()
        pltpu.make_async_copy(p2_hbm.at[iv.at[1]], bufB, sg).wait()
        pltpu.make_async_copy(c_hbm.at[pl.ds(0, CHUNK)], bufC, sg).wait()
        if issue_next:
            _idx_drain(ivn, sin)
            _issue_gathers(j + 1, 1 - s)
        _process(bufA, bufB, bufC)
        pltpu.sync_copy(bufA, agg.at[iv.at[1]], add=True)
        if prefetch_idx:
            _idx_issue(j + 2, iv, si)

    _idx_issue(0, i0, si0)
    _idx_drain(i0, si0)
    _issue_gathers(0, 0)
    _idx_issue(1, i1, si1)

    def _pair(i, carry):
        j0 = 2 * i
        _handle(j0, 0, True, True)
        _handle(j0 + 1, 1, True, True)
        return carry
    lax.fori_loop(0, NCH // 2 - 1, _pair, 0)

    _handle(NCH - 2, 0, False, True)
    _handle(NCH - 1, 1, False, False)

    plsc.subcore_barrier()
    pltpu.sync_copy(agg.at[pl.ds(base, ROWS_PT)],
                    out_hbm.at[cid, pl.ds(base, ROWS_PT)])

    @pl.when(sid == NS - 1)
    def _flush_tail():
        pltpu.sync_copy(agg.at[pl.ds(NS * ROWS_PT, N - NS * ROWS_PT)],
                        out_hbm.at[cid, pl.ds(NS * ROWS_PT, N - NS * ROWS_PT)])


_sc_edges = functools.partial(
    pl.kernel,
    out_type=jax.ShapeDtypeStruct((NC, N, D), jnp.float32),
    mesh=plsc.VectorSubcoreMesh(core_axis_name="c", subcore_axis_name="s"),
    scratch_types=[
        pltpu.VMEM((2, CHUNK), jnp.int32),
        pltpu.VMEM((2, CHUNK), jnp.int32),
        pltpu.VMEM((CHUNK, D), jnp.float32),
        pltpu.VMEM((CHUNK, D), jnp.float32),
        pltpu.VMEM((CHUNK, D), jnp.float32),
        pltpu.VMEM((CHUNK, D), jnp.float32),
        pltpu.VMEM((CHUNK, D), jnp.float32),
        pltpu.VMEM((CHUNK, D), jnp.float32),
        pltpu.VMEM_SHARED((N, D), jnp.float32),
        pltpu.SemaphoreType.DMA,
        pltpu.SemaphoreType.DMA,
        pltpu.SemaphoreType.DMA,
        pltpu.SemaphoreType.DMA,
    ],
)(_sc_edges_body)


def kernel(x, edge_index, edge_attr, W_msg, b_msg, W_upd, b_upd):
    p1, p2 = pl.pallas_call(
        _proj_body,
        grid=(N // RB,),
        in_specs=[
            pl.BlockSpec((RB, D), lambda i: (i, 0)),
            pl.BlockSpec((D, D), lambda i: (0, 0)),
            pl.BlockSpec((D, D), lambda i: (1, 0)),
            pl.BlockSpec((D,), lambda i: (0,)),
        ],
        out_specs=[
            pl.BlockSpec((RB, D), lambda i: (i, 0)),
            pl.BlockSpec((RB, D), lambda i: (i, 0)),
        ],
        out_shape=[
            jax.ShapeDtypeStruct((N, D), jnp.float32),
            jax.ShapeDtypeStruct((N, D), jnp.float32),
        ],
    )(x, W_msg, W_msg, b_msg)

    c_edge = pl.pallas_call(
        _edge_term_body,
        grid=(E // EB,),
        in_specs=[
            pl.BlockSpec((EB, DE), lambda i: (i, 0)),
            pl.BlockSpec((8, D), lambda i: (2 * D // 8, 0)),
        ],
        out_specs=pl.BlockSpec((EB, D), lambda i: (i, 0)),
        out_shape=jax.ShapeDtypeStruct((E, D), jnp.float32),
    )(edge_attr, W_msg)

    parts = _sc_edges(p1, p2, edge_index.reshape(-1), c_edge)

    out = pl.pallas_call(
        _update_body,
        grid=(N // RB,),
        in_specs=[
            pl.BlockSpec((RB, D), lambda i: (i, 0)),
            pl.BlockSpec((NC, RB, D), lambda i: (0, i, 0)),
            pl.BlockSpec((D, D), lambda i: (0, 0)),
            pl.BlockSpec((D, D), lambda i: (1, 0)),
            pl.BlockSpec((D,), lambda i: (0,)),
        ],
        out_specs=pl.BlockSpec((RB, D), lambda i: (i, 0)),
        out_shape=jax.ShapeDtypeStruct((N, D), jnp.float32),
    )(x, parts, W_upd, W_upd, b_upd)
    return out
